# baseline (device time: 4427629 ns/iter reference)
import jax
import jax.numpy as jnp
from jax import lax
from jax.experimental import pallas as pl
from jax.experimental.pallas import tpu as pltpu

N_DEV = 4


def _ring_hop2(src_r, src_l):

    def body(sr_ref, sl_ref, or_ref, ol_ref, ss_r, rs_r, ss_l, rs_l):
        d = lax.axis_index("i")
        right = lax.rem(d + 1, N_DEV)
        left = lax.rem(d + N_DEV - 1, N_DEV)
        rdma_r = pltpu.make_async_remote_copy(
            src_ref=sr_ref, dst_ref=or_ref, send_sem=ss_r, recv_sem=rs_r,
            device_id=(right,), device_id_type=pl.DeviceIdType.MESH,
        )
        rdma_l = pltpu.make_async_remote_copy(
            src_ref=sl_ref, dst_ref=ol_ref, send_sem=ss_l, recv_sem=rs_l,
            device_id=(left,), device_id_type=pl.DeviceIdType.MESH,
        )
        rdma_r.start()
        rdma_l.start()
        rdma_r.wait()
        rdma_l.wait()

    return pl.pallas_call(
        body,
        out_shape=(
            jax.ShapeDtypeStruct(src_r.shape, src_r.dtype),
            jax.ShapeDtypeStruct(src_l.shape, src_l.dtype),
        ),
        in_specs=[
            pl.BlockSpec(memory_space=pl.ANY),
            pl.BlockSpec(memory_space=pl.ANY),
        ],
        out_specs=(
            pl.BlockSpec(memory_space=pl.ANY),
            pl.BlockSpec(memory_space=pl.ANY),
        ),
        scratch_shapes=[pltpu.SemaphoreType.DMA] * 4,
    )(src_r, src_l)


def _all_gather_into_out(own_a, own_b):
    m, half = own_a.shape

    def body(oa_ref, ob_ref, out_ref, stage_a, stage_b, loc_sems, sa, ra, sb, rb):
        d = lax.axis_index("i")
        right = lax.rem(d + 1, N_DEV)
        left = lax.rem(d + N_DEV - 1, N_DEV)
        copies = []

        cp_a = pltpu.make_async_copy(
            oa_ref, out_ref.at[pl.ds(d * m, m), pl.ds(0, half)], loc_sems.at[0]
        )
        cp_b = pltpu.make_async_copy(
            ob_ref, out_ref.at[pl.ds(d * m, m), pl.ds(half, half)], loc_sems.at[1]
        )
        cp_a.start()
        cp_b.start()
        copies += [cp_a, cp_b]

        src_a, src_b = oa_ref, ob_ref
        for s in range(N_DEV - 1):
            rdma_a = pltpu.make_async_remote_copy(
                src_ref=src_a, dst_ref=stage_a.at[s],
                send_sem=sa.at[s], recv_sem=ra.at[s],
                device_id=(right,), device_id_type=pl.DeviceIdType.MESH,
            )
            rdma_b = pltpu.make_async_remote_copy(
                src_ref=src_b, dst_ref=stage_b.at[s],
                send_sem=sb.at[s], recv_sem=rb.at[s],
                device_id=(left,), device_id_type=pl.DeviceIdType.MESH,
            )
            rdma_a.start()
            rdma_b.start()
            rdma_a.wait()
            rdma_b.wait()
            ia = lax.rem(d + 2 * N_DEV - 1 - s, N_DEV)
            ib = lax.rem(d + 1 + s, N_DEV)
            cpa = pltpu.make_async_copy(
                stage_a.at[s],
                out_ref.at[pl.ds(ia * m, m), pl.ds(0, half)],
                loc_sems.at[2 + 2 * s],
            )
            cpb = pltpu.make_async_copy(
                stage_b.at[s],
                out_ref.at[pl.ds(ib * m, m), pl.ds(half, half)],
                loc_sems.at[3 + 2 * s],
            )
            cpa.start()
            cpb.start()
            copies += [cpa, cpb]
            src_a, src_b = stage_a.at[s], stage_b.at[s]

        for cp in copies:
            cp.wait()

    out, _, _ = pl.pallas_call(
        body,
        out_shape=(
            jax.ShapeDtypeStruct((N_DEV * m, 2 * half), own_a.dtype),
            jax.ShapeDtypeStruct((N_DEV - 1, m, half), own_a.dtype),
            jax.ShapeDtypeStruct((N_DEV - 1, m, half), own_a.dtype),
        ),
        in_specs=[
            pl.BlockSpec(memory_space=pl.ANY),
            pl.BlockSpec(memory_space=pl.ANY),
        ],
        out_specs=(
            pl.BlockSpec(memory_space=pl.ANY),
            pl.BlockSpec(memory_space=pl.ANY),
            pl.BlockSpec(memory_space=pl.ANY),
        ),
        scratch_shapes=[
            pltpu.SemaphoreType.DMA((2 * N_DEV,)),
            pltpu.SemaphoreType.DMA((N_DEV - 1,)),
            pltpu.SemaphoreType.DMA((N_DEV - 1,)),
            pltpu.SemaphoreType.DMA((N_DEV - 1,)),
            pltpu.SemaphoreType.DMA((N_DEV - 1,)),
        ],
    )(own_a, own_b)
    return out


def kernel(x, w_mat, scale_x, scale_w):
    d = lax.axis_index("i")

    xb = x.astype(jnp.bfloat16)
    wb = w_mat.astype(jnp.bfloat16)
    half = wb.shape[1] // 2
    pa = jnp.dot(xb, wb[:, :half], preferred_element_type=jnp.float32)
    pb = jnp.dot(xb, wb[:, half:], preferred_element_type=jnp.float32)

    m_tot = pa.shape[0]
    m = m_tot // N_DEV

    def chunk(p, i):
        return lax.dynamic_slice_in_dim(p, lax.rem(i, N_DEV) * m, m, axis=0)

    cur_a = chunk(pa, d + (N_DEV - 1))
    cur_b = chunk(pb, d + 1)
    for s in range(N_DEV - 1):
        ra, rb = _ring_hop2(cur_a, cur_b)
        cur_a = ra + chunk(pa, d + (2 * N_DEV - 2 - s))
        cur_b = rb + chunk(pb, d + 2 + s)

    scale = scale_x[0] * scale_w[0]

    def silu(acc):
        y = acc * scale
        return y * (1.0 / (1.0 + jnp.exp(-jnp.clip(y, -60.0, 60.0))))

    own_a = silu(cur_a)
    own_b = silu(cur_b)

    return _all_gather_into_out(own_a, own_b)


# device time: 2439077 ns/iter; 1.8153x vs baseline; 1.8153x over previous
import jax
import jax.numpy as jnp
from jax import lax
from jax.experimental import pallas as pl
from jax.experimental.pallas import tpu as pltpu

N_DEV = 4


def _ring_hop2(src_r, src_l):

    def body(sr_ref, sl_ref, or_ref, ol_ref, ss_r, rs_r, ss_l, rs_l):
        d = lax.axis_index("i")
        right = lax.rem(d + 1, N_DEV)
        left = lax.rem(d + N_DEV - 1, N_DEV)
        rdma_r = pltpu.make_async_remote_copy(
            src_ref=sr_ref, dst_ref=or_ref, send_sem=ss_r, recv_sem=rs_r,
            device_id=(right,), device_id_type=pl.DeviceIdType.MESH,
        )
        rdma_l = pltpu.make_async_remote_copy(
            src_ref=sl_ref, dst_ref=ol_ref, send_sem=ss_l, recv_sem=rs_l,
            device_id=(left,), device_id_type=pl.DeviceIdType.MESH,
        )
        rdma_r.start()
        rdma_l.start()
        rdma_r.wait()
        rdma_l.wait()

    return pl.pallas_call(
        body,
        out_shape=(
            jax.ShapeDtypeStruct(src_r.shape, src_r.dtype),
            jax.ShapeDtypeStruct(src_l.shape, src_l.dtype),
        ),
        in_specs=[
            pl.BlockSpec(memory_space=pl.ANY),
            pl.BlockSpec(memory_space=pl.ANY),
        ],
        out_specs=(
            pl.BlockSpec(memory_space=pl.ANY),
            pl.BlockSpec(memory_space=pl.ANY),
        ),
        scratch_shapes=[pltpu.SemaphoreType.DMA] * 4,
    )(src_r, src_l)


def _all_gather_into_out(own):
    m, n = own.shape
    h = m // 2

    def body(own_ref, out_ref, loc_sem, sa, ra, sb, rb):
        d = lax.axis_index("i")
        right = lax.rem(d + 1, N_DEV)
        left = lax.rem(d + N_DEV - 1, N_DEV)

        cp = pltpu.make_async_copy(
            own_ref, out_ref.at[pl.ds(d * m, m), :], loc_sem
        )
        cp.start()
        cp.wait()

        for s in range(N_DEV - 1):
            ia = lax.rem(d + N_DEV - s, N_DEV)
            ib = lax.rem(d + s, N_DEV)
            sl_a = (pl.ds(ia * m, h), slice(None))
            sl_b = (pl.ds(ib * m + h, h), slice(None))
            rdma_a = pltpu.make_async_remote_copy(
                src_ref=out_ref.at[sl_a], dst_ref=out_ref.at[sl_a],
                send_sem=sa.at[s], recv_sem=ra.at[s],
                device_id=(right,), device_id_type=pl.DeviceIdType.MESH,
            )
            rdma_b = pltpu.make_async_remote_copy(
                src_ref=out_ref.at[sl_b], dst_ref=out_ref.at[sl_b],
                send_sem=sb.at[s], recv_sem=rb.at[s],
                device_id=(left,), device_id_type=pl.DeviceIdType.MESH,
            )
            rdma_a.start()
            rdma_b.start()
            rdma_a.wait()
            rdma_b.wait()

    return pl.pallas_call(
        body,
        out_shape=jax.ShapeDtypeStruct((N_DEV * m, n), own.dtype),
        in_specs=[pl.BlockSpec(memory_space=pl.ANY)],
        out_specs=pl.BlockSpec(memory_space=pl.ANY),
        scratch_shapes=[
            pltpu.SemaphoreType.DMA,
            pltpu.SemaphoreType.DMA((N_DEV - 1,)),
            pltpu.SemaphoreType.DMA((N_DEV - 1,)),
            pltpu.SemaphoreType.DMA((N_DEV - 1,)),
            pltpu.SemaphoreType.DMA((N_DEV - 1,)),
        ],
    )(own)


def kernel(x, w_mat, scale_x, scale_w):
    d = lax.axis_index("i")

    partial = jnp.dot(
        x.astype(jnp.bfloat16),
        w_mat.astype(jnp.bfloat16),
        preferred_element_type=jnp.float32,
    )

    m_tot, n = partial.shape
    m = m_tot // N_DEV
    h = m // 2

    def upper(i):
        return lax.dynamic_slice_in_dim(
            partial, lax.rem(i, N_DEV) * m, h, axis=0
        )

    def lower(i):
        return lax.dynamic_slice_in_dim(
            partial, lax.rem(i, N_DEV) * m + h, h, axis=0
        )

    cur_a = upper(d + (N_DEV - 1))
    cur_b = lower(d + 1)
    for s in range(N_DEV - 1):
        ra, rb = _ring_hop2(cur_a, cur_b)
        cur_a = ra + upper(d + (2 * N_DEV - 2 - s))
        cur_b = rb + lower(d + 2 + s)

    scale = scale_x[0] * scale_w[0]

    def silu(acc):
        y = acc * scale
        return y * (1.0 / (1.0 + jnp.exp(-jnp.clip(y, -60.0, 60.0))))

    own = jnp.concatenate([silu(cur_a), silu(cur_b)], axis=0)

    return _all_gather_into_out(own)
